# baseline (speedup 1.0000x reference)
import functools

import jax
import jax.numpy as jnp
from jax import lax
from jax.experimental import pallas as pl
from jax.experimental.pallas import tpu as pltpu, tpu_sc as plsc

_SCALE = 10 - 0.0001
_N = 4194304
_NW = 32
_PER_W = _N // _NW
_C = 8192
_CHUNKS = _PER_W // _C
_L = 16


def _sc_body(x_hbm, t_hbm, o_hbm, xb, tb, ob, semx, semt):
    wid = lax.axis_index("s") * 2 + lax.axis_index("c")
    base = wid * _PER_W

    def chunk_body(ci, _):
        off = base + ci * _C
        cpx = pltpu.async_copy(x_hbm.at[pl.ds(off, _C)], xb, semx)
        cpt = pltpu.async_copy(t_hbm.at[pl.ds(off, _C)], tb, semt)
        cpx.wait()
        cpt.wait()

        def vec_body(i, _):
            s = i * _L
            xv = xb[pl.ds(s, _L)]
            tv = tb[pl.ds(s, _L)]
            sig = 1.0 / (1.0 + jnp.exp(-xv))
            g = jnp.abs(sig - tv)
            ob[pl.ds(s, _L)] = (g * _SCALE).astype(jnp.int32)
            return 0

        lax.fori_loop(0, _C // _L, vec_body, 0, unroll=4)
        pltpu.sync_copy(ob, o_hbm.at[pl.ds(off, _C)])
        return 0

    lax.fori_loop(0, _CHUNKS, chunk_body, 0)


@jax.jit
def kernel(x, target):
    mesh = plsc.VectorSubcoreMesh(core_axis_name="c", subcore_axis_name="s")
    run = functools.partial(
        pl.kernel,
        mesh=mesh,
        out_type=jax.ShapeDtypeStruct((_N,), jnp.int32),
        scratch_types=[
            pltpu.VMEM((_C,), jnp.float32),
            pltpu.VMEM((_C,), jnp.float32),
            pltpu.VMEM((_C,), jnp.int32),
            pltpu.SemaphoreType.DMA,
            pltpu.SemaphoreType.DMA,
        ],
    )(_sc_body)
    return run(x, target)


# speedup vs baseline: 5.5034x; 5.5034x over previous
import functools

import jax
import jax.numpy as jnp
from jax import lax
from jax.experimental import pallas as pl
from jax.experimental.pallas import tpu as pltpu, tpu_sc as plsc

_SCALE = 10 - 0.0001
_N = 4194304
_NW = 32
_PER_W = _N // _NW
_C = 16384
_CHUNKS = _PER_W // _C
_L = 16
_UNROLL = 8


def _sc_body(x_hbm, t_hbm, o_hbm,
             xb0, xb1, tb0, tb1, ob0, ob1,
             sx0, sx1, st0, st1, so0, so1):
    wid = lax.axis_index("s") * 2 + lax.axis_index("c")
    base = wid * _PER_W
    xbufs, tbufs, obufs = (xb0, xb1), (tb0, tb1), (ob0, ob1)
    sxs, sts, sos = (sx0, sx1), (st0, st1), (so0, so1)

    def start_in(ci):
        b = ci % 2
        off = base + ci * _C
        cx = pltpu.async_copy(x_hbm.at[pl.ds(off, _C)], xbufs[b], sxs[b])
        ct = pltpu.async_copy(t_hbm.at[pl.ds(off, _C)], tbufs[b], sts[b])
        return cx, ct

    pend_in = start_in(0)
    pend_out = [None, None]
    for ci in range(_CHUNKS):
        b = ci % 2
        xb, tb, ob = xbufs[b], tbufs[b], obufs[b]
        cx, ct = pend_in
        cx.wait()
        ct.wait()
        if ci + 1 < _CHUNKS:
            pend_in = start_in(ci + 1)
        if pend_out[b] is not None:
            pend_out[b].wait()

        @plsc.parallel_loop(0, _C, step=_L, unroll=_UNROLL)
        def _compute(s):
            xv = xb[pl.ds(s, _L)]
            tv = tb[pl.ds(s, _L)]
            sig = 1.0 / (1.0 + jnp.exp(-xv))
            g = jnp.abs(sig - tv)
            ob[pl.ds(s, _L)] = (g * _SCALE).astype(jnp.int32)

        pend_out[b] = pltpu.async_copy(
            ob, o_hbm.at[pl.ds(base + ci * _C, _C)], sos[b])
    for b in range(2):
        if pend_out[b] is not None:
            pend_out[b].wait()


@jax.jit
def kernel(x, target):
    mesh = plsc.VectorSubcoreMesh(core_axis_name="c", subcore_axis_name="s")
    run = functools.partial(
        pl.kernel,
        mesh=mesh,
        out_type=jax.ShapeDtypeStruct((_N,), jnp.int32),
        scratch_types=[
            pltpu.VMEM((_C,), jnp.float32),
            pltpu.VMEM((_C,), jnp.float32),
            pltpu.VMEM((_C,), jnp.float32),
            pltpu.VMEM((_C,), jnp.float32),
            pltpu.VMEM((_C,), jnp.int32),
            pltpu.VMEM((_C,), jnp.int32),
            pltpu.SemaphoreType.DMA,
            pltpu.SemaphoreType.DMA,
            pltpu.SemaphoreType.DMA,
            pltpu.SemaphoreType.DMA,
            pltpu.SemaphoreType.DMA,
            pltpu.SemaphoreType.DMA,
        ],
    )(_sc_body)
    return run(x, target)


# speedup vs baseline: 5.8350x; 1.0603x over previous
import functools

import jax
import jax.numpy as jnp
from jax import lax
from jax.experimental import pallas as pl
from jax.experimental.pallas import tpu as pltpu, tpu_sc as plsc

_SCALE = 10 - 0.0001
_NEG_LOG2E = -1.4426950408889634
_N = 4194304
_NW = 32
_PER_W = _N // _NW
_C = 16384
_CHUNKS = _PER_W // _C
_L = 16
_UNROLL = 8


def _sc_body(x_hbm, t_hbm, o_hbm,
             xb0, xb1, tb0, tb1, ob0, ob1,
             sx0, sx1, st0, st1, so0, so1):
    wid = lax.axis_index("s") * 2 + lax.axis_index("c")
    base = wid * _PER_W
    xbufs, tbufs, obufs = (xb0, xb1), (tb0, tb1), (ob0, ob1)
    sxs, sts, sos = (sx0, sx1), (st0, st1), (so0, so1)

    def start_in(c, b):
        off = base + c * _C
        pltpu.async_copy(x_hbm.at[pl.ds(off, _C)], xbufs[b], sxs[b])
        pltpu.async_copy(t_hbm.at[pl.ds(off, _C)], tbufs[b], sts[b])

    start_in(0, 0)
    start_in(1, 1)

    @pl.loop(0, _CHUNKS // 2)
    def _chunks(g):
        for b in range(2):
            c = g * 2 + b
            xb, tb, ob = xbufs[b], tbufs[b], obufs[b]
            pltpu.make_async_copy(x_hbm.at[pl.ds(0, _C)], xb, sxs[b]).wait()
            pltpu.make_async_copy(t_hbm.at[pl.ds(0, _C)], tb, sts[b]).wait()
            @pl.when(g >= 1)
            def _():
                pltpu.make_async_copy(
                    ob, o_hbm.at[pl.ds(base, _C)], sos[b]).wait()

            @plsc.parallel_loop(0, _C, step=_L, unroll=_UNROLL)
            def _compute(s):
                xv = xb[pl.ds(s, _L)]
                tv = tb[pl.ds(s, _L)]
                sig = 1.0 / (1.0 + jnp.exp(xv * -1.0))
                g_ = jnp.abs(sig - tv)
                ob[pl.ds(s, _L)] = (g_ * _SCALE).astype(jnp.int32)

            pltpu.async_copy(ob, o_hbm.at[pl.ds(base + c * _C, _C)], sos[b])
            @pl.when(c + 2 < _CHUNKS)
            def _():
                start_in(c + 2, b)

    for b in range(2):
        pltpu.make_async_copy(obufs[b], o_hbm.at[pl.ds(base, _C)],
                              sos[b]).wait()


@jax.jit
def kernel(x, target):
    mesh = plsc.VectorSubcoreMesh(core_axis_name="c", subcore_axis_name="s")
    run = functools.partial(
        pl.kernel,
        mesh=mesh,
        compiler_params=pltpu.CompilerParams(use_tc_tiling_on_sc=True),
        out_type=jax.ShapeDtypeStruct((_N,), jnp.int32),
        scratch_types=[
            pltpu.VMEM((_C,), jnp.float32),
            pltpu.VMEM((_C,), jnp.float32),
            pltpu.VMEM((_C,), jnp.float32),
            pltpu.VMEM((_C,), jnp.float32),
            pltpu.VMEM((_C,), jnp.int32),
            pltpu.VMEM((_C,), jnp.int32),
            pltpu.SemaphoreType.DMA,
            pltpu.SemaphoreType.DMA,
            pltpu.SemaphoreType.DMA,
            pltpu.SemaphoreType.DMA,
            pltpu.SemaphoreType.DMA,
            pltpu.SemaphoreType.DMA,
        ],
    )(_sc_body)
    return run(x, target)
